# baseline (speedup 1.0000x reference)
import functools

import jax
import jax.numpy as jnp
from jax import lax
from jax.experimental import pallas as pl
from jax.experimental.pallas import tpu as pltpu
from jax.experimental.pallas import tpu_sc as plsc

_B, _N, _IN_DIM, _HID, _CODE_DIM, _N_CODES = 32, 256, 768, 512, 256, 8192
_ROWS = _B * _N
_M = 512
_JB = 2048
_R = _ROWS // _M
_J = _N_CODES // _JB

_NC, _NS = 2, 16
_NW = _NC * _NS
_BPW = _ROWS // _NW


def _sweep(s_src_ref, bv_ref, bc_ref, tok_ref, t):
    jp = (t - 1) % _J
    nchunk = _JB // 8
    ngroups = 4
    per = nchunk // ngroups
    s = s_src_ref[...]
    gv = [jnp.full((8, _M), jnp.inf, jnp.float32)] * ngroups
    gc = [jnp.zeros((8, _M), jnp.int32)] * ngroups
    for c in range(nchunk):
        g = c // per
        sc = lax.slice(s, (c * 8, 0), (c * 8 + 8, _M))
        cmp = sc < gv[g]
        gv[g] = jnp.where(cmp, sc, gv[g])
        gc[g] = jnp.where(cmp, c, gc[g])
    bv = jnp.where(jp == 0, jnp.full((8, _M), jnp.inf, jnp.float32),
                   bv_ref[...])
    bc = jnp.where(jp == 0, jnp.zeros((8, _M), jnp.int32), bc_ref[...])
    off = jp * nchunk
    for g in range(ngroups):
        cmp = gv[g] < bv
        bv = jnp.where(cmp, gv[g], bv)
        bc = jnp.where(cmp, gc[g] + off, bc)
    bv_ref[...] = bv
    bc_ref[...] = bc

    @pl.when((t % _J == 0) & (t > 0))
    def _finalize():
        code = bc * 8 + lax.broadcasted_iota(jnp.int32, (8, _M), 0)
        lmin = jnp.min(bv, axis=0, keepdims=True)
        cand = jnp.where(bv == lmin, code, _N_CODES)
        tok_ref[...] = jnp.min(cand, axis=0, keepdims=True).reshape(1, 1, _M)


def _tc_body(x_ref, w1_ref, b1_ref, g1_ref, be1_ref, w2_ref, b2_ref, cb_ref,
             z_ref, tok_ref, zs_ref, bv_ref, bc_ref, sa_ref, sb_ref, cn_ref):
    t = pl.program_id(0)

    @pl.when((t % _J == 0) & (t < _R * _J))
    def _encode():
        h = jnp.dot(x_ref[...], w1_ref[...], preferred_element_type=jnp.float32)
        h = jnp.maximum(h + b1_ref[...], 0.0)
        mu = jnp.mean(h, axis=-1, keepdims=True)
        var = jnp.mean((h - mu) * (h - mu), axis=-1, keepdims=True)
        h = (h - mu) / jnp.sqrt(var + 1e-5) * g1_ref[...] + be1_ref[...]
        z = jnp.dot(h, w2_ref[...], preferred_element_type=jnp.float32)
        z = z + b2_ref[...]
        zs_ref[...] = -2.0 * z
        z_ref[...] = z

    @pl.when(t < _J)
    def _norms():
        cb = cb_ref[...]
        cn_ref[pl.ds(t * _JB, _JB), :] = jnp.sum(cb * cb, axis=1,
                                                 keepdims=True)

    def _scores():
        return lax.dot_general(
            cb_ref[...], zs_ref[...], (((1,), (1,)), ((), ())),
            preferred_element_type=jnp.float32
        ) + cn_ref[pl.ds((t % _J) * _JB, _JB), :]

    @pl.when(t % 2 == 0)
    def _even():
        sa_ref[...] = _scores()
        _sweep(sb_ref, bv_ref, bc_ref, tok_ref, t)

    @pl.when(t % 2 == 1)
    def _odd():
        sb_ref[...] = _scores()
        _sweep(sa_ref, bv_ref, bc_ref, tok_ref, t)


def _encode_quantize(x2d, w1, b1, g1, be1, w2, b2, codebook):
    grid = (_R * _J + 1,)
    z, tok = pl.pallas_call(
        _tc_body,
        grid=grid,
        in_specs=[
            pl.BlockSpec((_M, _IN_DIM),
                         lambda t: (jnp.minimum(t // _J, _R - 1), 0)),
            pl.BlockSpec((_IN_DIM, _HID), lambda t: (0, 0)),
            pl.BlockSpec((1, _HID), lambda t: (0, 0)),
            pl.BlockSpec((1, _HID), lambda t: (0, 0)),
            pl.BlockSpec((1, _HID), lambda t: (0, 0)),
            pl.BlockSpec((_HID, _CODE_DIM), lambda t: (0, 0)),
            pl.BlockSpec((1, _CODE_DIM), lambda t: (0, 0)),
            pl.BlockSpec((_JB, _CODE_DIM), lambda t: (t % _J, 0)),
        ],
        out_specs=[
            pl.BlockSpec((_M, _CODE_DIM),
                         lambda t: (jnp.minimum(t // _J, _R - 1), 0)),
            pl.BlockSpec((1, 1, _M),
                         lambda t: (jnp.maximum(t - 1, 0) // _J, 0, 0)),
        ],
        out_shape=[
            jax.ShapeDtypeStruct((_ROWS, _CODE_DIM), jnp.float32),
            jax.ShapeDtypeStruct((_R, 1, _M), jnp.int32),
        ],
        scratch_shapes=[
            pltpu.VMEM((_M, _CODE_DIM), jnp.float32),
            pltpu.VMEM((8, _M), jnp.float32),
            pltpu.VMEM((8, _M), jnp.int32),
            pltpu.VMEM((_JB, _M), jnp.float32),
            pltpu.VMEM((_JB, _M), jnp.float32),
            pltpu.VMEM((_N_CODES, 1), jnp.float32),
        ],
        compiler_params=pltpu.CompilerParams(
            dimension_semantics=("arbitrary",)),
    )(x2d, w1, b1, g1, be1, w2, b2, codebook)
    return z, tok.reshape(_ROWS)


@functools.cache
def _make_sc_gather():
    mesh = plsc.VectorSubcoreMesh(core_axis_name="c", subcore_axis_name="s")

    @functools.partial(
        pl.kernel,
        mesh=mesh,
        out_type=jax.ShapeDtypeStruct((_ROWS, _CODE_DIM), jnp.float32),
        scratch_types=[
            pltpu.VMEM((_BPW,), jnp.int32),
            pltpu.VMEM((_BPW, _CODE_DIM), jnp.float32),
            pltpu.SemaphoreType.DMA,
        ],
    )
    def _sc_gather(cb_hbm, idx_hbm, out_hbm, idx_v, rows_v, sem):
        wid = lax.axis_index("s") * _NC + lax.axis_index("c")
        base = wid * _BPW
        pltpu.sync_copy(idx_hbm.at[pl.ds(base, _BPW)], idx_v)
        pltpu.async_copy(cb_hbm.at[idx_v], rows_v, sem).wait()
        pltpu.sync_copy(rows_v, out_hbm.at[pl.ds(base, _BPW)])

    return _sc_gather


def kernel(x, W1, b1, g1, be1, W2, b2, codebook):
    x2d = x.reshape(_ROWS, _IN_DIM)
    z_flat, tokens = _encode_quantize(
        x2d, W1, b1.reshape(1, _HID), g1.reshape(1, _HID),
        be1.reshape(1, _HID), W2, b2.reshape(1, _CODE_DIM), codebook)
    z_q = _make_sc_gather()(codebook, tokens)
    emb = z_flat + (z_q - z_flat)
    return (tokens.reshape(_B, _N),
            emb.reshape(_B, _N, _CODE_DIM),
            z_flat.reshape(_B, _N, _CODE_DIM))


# speedup vs baseline: 1.3172x; 1.3172x over previous
import functools

import jax
import jax.numpy as jnp
from jax import lax
from jax.experimental import pallas as pl
from jax.experimental.pallas import tpu as pltpu
from jax.experimental.pallas import tpu_sc as plsc

_B, _N, _IN_DIM, _HID, _CODE_DIM, _N_CODES = 32, 256, 768, 512, 256, 8192
_ROWS = _B * _N
_M = 1024
_JB = 2048
_R = _ROWS // _M
_J = _N_CODES // _JB

_NC, _NS = 2, 16
_NW = _NC * _NS
_BPW = _ROWS // _NW


def _sweep(s_src_ref, bv_ref, bc_ref, tok_ref, t):
    jp = (t - 1) % _J
    nchunk = _JB // 8
    s = s_src_ref[...]
    bv = jnp.where(jp == 0, jnp.full((8, _M), jnp.inf, jnp.float32),
                   bv_ref[...])
    bc = jnp.where(jp == 0, jnp.zeros((8, _M), jnp.int32), bc_ref[...])
    for c in range(nchunk):
        sc = lax.slice(s, (c * 8, 0), (c * 8 + 8, _M))
        cmp = sc < bv
        bv = jnp.where(cmp, sc, bv)
        bc = jnp.where(cmp, jp * nchunk + c, bc)
    bv_ref[...] = bv
    bc_ref[...] = bc

    @pl.when((t % _J == 0) & (t > 0))
    def _finalize():
        code = bc * 8 + lax.broadcasted_iota(jnp.int32, (8, _M), 0)
        lmin = jnp.min(bv, axis=0, keepdims=True)
        cand = jnp.where(bv == lmin, code, _N_CODES)
        tok_ref[...] = jnp.min(cand, axis=0, keepdims=True).reshape(1, 1, _M)


def _tc_body(x_ref, w1_ref, b1_ref, g1_ref, be1_ref, w2_ref, b2_ref, cb_ref,
             z_ref, tok_ref, zs_ref, bv_ref, bc_ref, sa_ref, sb_ref, cn_ref):
    t = pl.program_id(0)

    @pl.when((t % _J == 0) & (t < _R * _J))
    def _encode():
        h = jnp.dot(x_ref[...], w1_ref[...], preferred_element_type=jnp.float32)
        h = jnp.maximum(h + b1_ref[...], 0.0)
        mu = jnp.mean(h, axis=-1, keepdims=True)
        var = jnp.mean((h - mu) * (h - mu), axis=-1, keepdims=True)
        h = (h - mu) / jnp.sqrt(var + 1e-5) * g1_ref[...] + be1_ref[...]
        z = jnp.dot(h, w2_ref[...], preferred_element_type=jnp.float32)
        z = z + b2_ref[...]
        zs_ref[...] = -2.0 * z
        z_ref[...] = z

    @pl.when(t < _J)
    def _norms():
        cb = cb_ref[...]
        cn_ref[pl.ds(t * _JB, _JB), :] = jnp.sum(cb * cb, axis=1,
                                                 keepdims=True)

    def _scores():
        return lax.dot_general(
            cb_ref[...], zs_ref[...], (((1,), (1,)), ((), ())),
            preferred_element_type=jnp.float32
        ) + cn_ref[pl.ds((t % _J) * _JB, _JB), :]

    @pl.when(t % 2 == 0)
    def _even():
        sa_ref[...] = _scores()
        _sweep(sb_ref, bv_ref, bc_ref, tok_ref, t)

    @pl.when(t % 2 == 1)
    def _odd():
        sb_ref[...] = _scores()
        _sweep(sa_ref, bv_ref, bc_ref, tok_ref, t)


def _encode_quantize(x2d, w1, b1, g1, be1, w2, b2, codebook):
    grid = (_R * _J + 1,)
    z, tok = pl.pallas_call(
        _tc_body,
        grid=grid,
        in_specs=[
            pl.BlockSpec((_M, _IN_DIM),
                         lambda t: (jnp.minimum(t // _J, _R - 1), 0)),
            pl.BlockSpec((_IN_DIM, _HID), lambda t: (0, 0)),
            pl.BlockSpec((1, _HID), lambda t: (0, 0)),
            pl.BlockSpec((1, _HID), lambda t: (0, 0)),
            pl.BlockSpec((1, _HID), lambda t: (0, 0)),
            pl.BlockSpec((_HID, _CODE_DIM), lambda t: (0, 0)),
            pl.BlockSpec((1, _CODE_DIM), lambda t: (0, 0)),
            pl.BlockSpec((_JB, _CODE_DIM), lambda t: (t % _J, 0)),
        ],
        out_specs=[
            pl.BlockSpec((_M, _CODE_DIM),
                         lambda t: (jnp.minimum(t // _J, _R - 1), 0)),
            pl.BlockSpec((1, 1, _M),
                         lambda t: (jnp.maximum(t - 1, 0) // _J, 0, 0)),
        ],
        out_shape=[
            jax.ShapeDtypeStruct((_ROWS, _CODE_DIM), jnp.float32),
            jax.ShapeDtypeStruct((_R, 1, _M), jnp.int32),
        ],
        scratch_shapes=[
            pltpu.VMEM((_M, _CODE_DIM), jnp.float32),
            pltpu.VMEM((8, _M), jnp.float32),
            pltpu.VMEM((8, _M), jnp.int32),
            pltpu.VMEM((_JB, _M), jnp.float32),
            pltpu.VMEM((_JB, _M), jnp.float32),
            pltpu.VMEM((_N_CODES, 1), jnp.float32),
        ],
        compiler_params=pltpu.CompilerParams(
            dimension_semantics=("arbitrary",)),
    )(x2d, w1, b1, g1, be1, w2, b2, codebook)
    return z, tok.reshape(_ROWS)


@functools.cache
def _make_sc_gather():
    mesh = plsc.VectorSubcoreMesh(core_axis_name="c", subcore_axis_name="s")

    @functools.partial(
        pl.kernel,
        mesh=mesh,
        out_type=jax.ShapeDtypeStruct((_ROWS, _CODE_DIM), jnp.float32),
        scratch_types=[
            pltpu.VMEM((_BPW,), jnp.int32),
            pltpu.VMEM((_BPW, _CODE_DIM), jnp.float32),
            pltpu.SemaphoreType.DMA,
        ],
    )
    def _sc_gather(cb_hbm, idx_hbm, out_hbm, idx_v, rows_v, sem):
        wid = lax.axis_index("s") * _NC + lax.axis_index("c")
        base = wid * _BPW
        pltpu.sync_copy(idx_hbm.at[pl.ds(base, _BPW)], idx_v)
        pltpu.async_copy(cb_hbm.at[idx_v], rows_v, sem).wait()
        pltpu.sync_copy(rows_v, out_hbm.at[pl.ds(base, _BPW)])

    return _sc_gather


def kernel(x, W1, b1, g1, be1, W2, b2, codebook):
    x2d = x.reshape(_ROWS, _IN_DIM)
    z_flat, tokens = _encode_quantize(
        x2d, W1, b1.reshape(1, _HID), g1.reshape(1, _HID),
        be1.reshape(1, _HID), W2, b2.reshape(1, _CODE_DIM), codebook)
    z_q = _make_sc_gather()(codebook, tokens)
    emb = z_flat + (z_q - z_flat)
    return (tokens.reshape(_B, _N),
            emb.reshape(_B, _N, _CODE_DIM),
            z_flat.reshape(_B, _N, _CODE_DIM))


# speedup vs baseline: 1.3667x; 1.0375x over previous
import functools

import jax
import jax.numpy as jnp
from jax import lax
from jax.experimental import pallas as pl
from jax.experimental.pallas import tpu as pltpu
from jax.experimental.pallas import tpu_sc as plsc

_B, _N, _IN_DIM, _HID, _CODE_DIM, _N_CODES = 32, 256, 768, 512, 256, 8192
_ROWS = _B * _N
_M = 1024
_JB = 4096
_R = _ROWS // _M
_J = _N_CODES // _JB

_NC, _NS = 2, 16
_NW = _NC * _NS
_BPW = _ROWS // _NW


def _sweep(s_src_ref, bv_ref, bc_ref, tok_ref, t):
    jp = (t - 1) % _J
    nchunk = _JB // 8
    s = s_src_ref[...]
    bv = jnp.where(jp == 0, jnp.full((8, _M), jnp.inf, jnp.float32),
                   bv_ref[...])
    bc = jnp.where(jp == 0, jnp.zeros((8, _M), jnp.int32), bc_ref[...])
    for c in range(nchunk):
        sc = lax.slice(s, (c * 8, 0), (c * 8 + 8, _M))
        cmp = sc < bv
        bv = jnp.where(cmp, sc, bv)
        bc = jnp.where(cmp, jp * nchunk + c, bc)
    bv_ref[...] = bv
    bc_ref[...] = bc

    @pl.when((t % _J == 0) & (t > 0))
    def _finalize():
        code = bc * 8 + lax.broadcasted_iota(jnp.int32, (8, _M), 0)
        lmin = jnp.min(bv, axis=0, keepdims=True)
        cand = jnp.where(bv == lmin, code, _N_CODES)
        tok_ref[...] = jnp.min(cand, axis=0, keepdims=True).reshape(1, 1, _M)


def _tc_body(x_ref, w1_ref, b1_ref, g1_ref, be1_ref, w2_ref, b2_ref, cb_ref,
             z_ref, tok_ref, zs_ref, bv_ref, bc_ref, sa_ref, sb_ref, cn_ref):
    t = pl.program_id(0)

    @pl.when((t % _J == 0) & (t < _R * _J))
    def _encode():
        h = jnp.dot(x_ref[...], w1_ref[...], preferred_element_type=jnp.float32)
        h = jnp.maximum(h + b1_ref[...], 0.0)
        mu = jnp.mean(h, axis=-1, keepdims=True)
        var = jnp.mean((h - mu) * (h - mu), axis=-1, keepdims=True)
        h = (h - mu) / jnp.sqrt(var + 1e-5) * g1_ref[...] + be1_ref[...]
        z = jnp.dot(h, w2_ref[...], preferred_element_type=jnp.float32)
        z = z + b2_ref[...]
        zs_ref[...] = -2.0 * z
        z_ref[...] = z

    @pl.when(t < _J)
    def _norms():
        cb = cb_ref[...]
        cn_ref[pl.ds(t * _JB, _JB), :] = jnp.sum(cb * cb, axis=1,
                                                 keepdims=True)

    def _scores():
        return lax.dot_general(
            cb_ref[...], zs_ref[...], (((1,), (1,)), ((), ())),
            preferred_element_type=jnp.float32
        ) + cn_ref[pl.ds((t % _J) * _JB, _JB), :]

    @pl.when(t % 2 == 0)
    def _even():
        sa_ref[...] = _scores()
        _sweep(sb_ref, bv_ref, bc_ref, tok_ref, t)

    @pl.when(t % 2 == 1)
    def _odd():
        sb_ref[...] = _scores()
        _sweep(sa_ref, bv_ref, bc_ref, tok_ref, t)


def _encode_quantize(x2d, w1, b1, g1, be1, w2, b2, codebook):
    grid = (_R * _J + 1,)
    z, tok = pl.pallas_call(
        _tc_body,
        grid=grid,
        in_specs=[
            pl.BlockSpec((_M, _IN_DIM),
                         lambda t: (jnp.minimum(t // _J, _R - 1), 0)),
            pl.BlockSpec((_IN_DIM, _HID), lambda t: (0, 0)),
            pl.BlockSpec((1, _HID), lambda t: (0, 0)),
            pl.BlockSpec((1, _HID), lambda t: (0, 0)),
            pl.BlockSpec((1, _HID), lambda t: (0, 0)),
            pl.BlockSpec((_HID, _CODE_DIM), lambda t: (0, 0)),
            pl.BlockSpec((1, _CODE_DIM), lambda t: (0, 0)),
            pl.BlockSpec((_JB, _CODE_DIM), lambda t: (t % _J, 0)),
        ],
        out_specs=[
            pl.BlockSpec((_M, _CODE_DIM),
                         lambda t: (jnp.minimum(t // _J, _R - 1), 0)),
            pl.BlockSpec((1, 1, _M),
                         lambda t: (jnp.maximum(t - 1, 0) // _J, 0, 0)),
        ],
        out_shape=[
            jax.ShapeDtypeStruct((_ROWS, _CODE_DIM), jnp.float32),
            jax.ShapeDtypeStruct((_R, 1, _M), jnp.int32),
        ],
        scratch_shapes=[
            pltpu.VMEM((_M, _CODE_DIM), jnp.float32),
            pltpu.VMEM((8, _M), jnp.float32),
            pltpu.VMEM((8, _M), jnp.int32),
            pltpu.VMEM((_JB, _M), jnp.float32),
            pltpu.VMEM((_JB, _M), jnp.float32),
            pltpu.VMEM((_N_CODES, 1), jnp.float32),
        ],
        compiler_params=pltpu.CompilerParams(
            dimension_semantics=("arbitrary",)),
    )(x2d, w1, b1, g1, be1, w2, b2, codebook)
    return z, tok.reshape(_ROWS)


@functools.cache
def _make_sc_gather():
    mesh = plsc.VectorSubcoreMesh(core_axis_name="c", subcore_axis_name="s")

    @functools.partial(
        pl.kernel,
        mesh=mesh,
        out_type=jax.ShapeDtypeStruct((_ROWS, _CODE_DIM), jnp.float32),
        scratch_types=[
            pltpu.VMEM((_BPW,), jnp.int32),
            pltpu.VMEM((_BPW, _CODE_DIM), jnp.float32),
            pltpu.SemaphoreType.DMA,
        ],
    )
    def _sc_gather(cb_hbm, idx_hbm, out_hbm, idx_v, rows_v, sem):
        wid = lax.axis_index("s") * _NC + lax.axis_index("c")
        base = wid * _BPW
        pltpu.sync_copy(idx_hbm.at[pl.ds(base, _BPW)], idx_v)
        pltpu.async_copy(cb_hbm.at[idx_v], rows_v, sem).wait()
        pltpu.sync_copy(rows_v, out_hbm.at[pl.ds(base, _BPW)])

    return _sc_gather


def kernel(x, W1, b1, g1, be1, W2, b2, codebook):
    x2d = x.reshape(_ROWS, _IN_DIM)
    z_flat, tokens = _encode_quantize(
        x2d, W1, b1.reshape(1, _HID), g1.reshape(1, _HID),
        be1.reshape(1, _HID), W2, b2.reshape(1, _CODE_DIM), codebook)
    z_q = _make_sc_gather()(codebook, tokens)
    emb = z_flat + (z_q - z_flat)
    return (tokens.reshape(_B, _N),
            emb.reshape(_B, _N, _CODE_DIM),
            z_flat.reshape(_B, _N, _CODE_DIM))


# speedup vs baseline: 1.4545x; 1.0643x over previous
import functools

import jax
import jax.numpy as jnp
from jax import lax
from jax.experimental import pallas as pl
from jax.experimental.pallas import tpu as pltpu
from jax.experimental.pallas import tpu_sc as plsc

_B, _N, _IN_DIM, _HID, _CODE_DIM, _N_CODES = 32, 256, 768, 512, 256, 8192
_ROWS = _B * _N
_M = 1024
_JB = 4096
_R = _ROWS // _M
_J = _N_CODES // _JB

_NC, _NS = 2, 16
_NW = _NC * _NS
_BPW = _ROWS // _NW


def _sweep(s_src_ref, bv_ref, bc_ref, tok_ref, t):
    jp = (t - 1) % _J
    nchunk = _JB // 8
    s = s_src_ref[...]
    bv = jnp.where(jp == 0, jnp.full((8, _M), jnp.inf, jnp.float32),
                   bv_ref[...])
    bc = jnp.where(jp == 0, jnp.zeros((8, _M), jnp.int32), bc_ref[...])
    for c in range(nchunk):
        sc = lax.slice(s, (c * 8, 0), (c * 8 + 8, _M))
        cmp = sc < bv
        bv = jnp.where(cmp, sc, bv)
        bc = jnp.where(cmp, jp * nchunk + c, bc)
    bv_ref[...] = bv
    bc_ref[...] = bc

    @pl.when((t % _J == 0) & (t > 0))
    def _finalize():
        code = bc * 8 + lax.broadcasted_iota(jnp.int32, (8, _M), 0)
        lmin = jnp.min(bv, axis=0, keepdims=True)
        cand = jnp.where(bv == lmin, code, _N_CODES)
        tok_ref[...] = jnp.min(cand, axis=0, keepdims=True).reshape(1, 1, _M)


def _tc_body(x_ref, w1_ref, b1_ref, g1_ref, be1_ref, w2_ref, b2_ref, cb_ref,
             z_ref, tok_ref, zs_ref, bv_ref, bc_ref, sa_ref, sb_ref, cn_ref):
    t = pl.program_id(0)

    @pl.when((t % _J == 0) & (t < _R * _J))
    def _encode():
        h = jnp.dot(x_ref[...], w1_ref[...], preferred_element_type=jnp.float32)
        h = jnp.maximum(h + b1_ref[...], 0.0)
        mu = jnp.mean(h, axis=-1, keepdims=True)
        var = jnp.mean((h - mu) * (h - mu), axis=-1, keepdims=True)
        h = (h - mu) / jnp.sqrt(var + 1e-5) * g1_ref[...] + be1_ref[...]
        z = jnp.dot(h, w2_ref[...], preferred_element_type=jnp.float32)
        z = z + b2_ref[...]
        zs_ref[...] = -2.0 * z
        z_ref[...] = z

    @pl.when(t < _J)
    def _norms():
        cb = cb_ref[...]
        cn_ref[pl.ds(t * _JB, _JB), :] = jnp.sum(cb * cb, axis=1,
                                                 keepdims=True)

    def _scores():
        return lax.dot_general(
            cb_ref[...], zs_ref[...], (((1,), (1,)), ((), ())),
            preferred_element_type=jnp.float32
        ) + cn_ref[pl.ds((t % _J) * _JB, _JB), :]

    @pl.when(t % 2 == 0)
    def _even():
        sa_ref[...] = _scores()
        _sweep(sb_ref, bv_ref, bc_ref, tok_ref, t)

    @pl.when(t % 2 == 1)
    def _odd():
        sb_ref[...] = _scores()
        _sweep(sa_ref, bv_ref, bc_ref, tok_ref, t)


def _encode_quantize(x2d, w1, b1, g1, be1, w2, b2, codebook):
    grid = (_R * _J + 1,)
    z, tok = pl.pallas_call(
        _tc_body,
        grid=grid,
        in_specs=[
            pl.BlockSpec((_M, _IN_DIM),
                         lambda t: (jnp.minimum(t // _J, _R - 1), 0)),
            pl.BlockSpec((_IN_DIM, _HID), lambda t: (0, 0)),
            pl.BlockSpec((1, _HID), lambda t: (0, 0)),
            pl.BlockSpec((1, _HID), lambda t: (0, 0)),
            pl.BlockSpec((1, _HID), lambda t: (0, 0)),
            pl.BlockSpec((_HID, _CODE_DIM), lambda t: (0, 0)),
            pl.BlockSpec((1, _CODE_DIM), lambda t: (0, 0)),
            pl.BlockSpec((_JB, _CODE_DIM), lambda t: (t % _J, 0)),
        ],
        out_specs=[
            pl.BlockSpec((_M, _CODE_DIM),
                         lambda t: (jnp.minimum(t // _J, _R - 1), 0)),
            pl.BlockSpec((1, 1, _M),
                         lambda t: (jnp.maximum(t - 1, 0) // _J, 0, 0)),
        ],
        out_shape=[
            jax.ShapeDtypeStruct((_ROWS, _CODE_DIM), jnp.float32),
            jax.ShapeDtypeStruct((_R, 1, _M), jnp.int32),
        ],
        scratch_shapes=[
            pltpu.VMEM((_M, _CODE_DIM), jnp.float32),
            pltpu.VMEM((8, _M), jnp.float32),
            pltpu.VMEM((8, _M), jnp.int32),
            pltpu.VMEM((_JB, _M), jnp.float32),
            pltpu.VMEM((_JB, _M), jnp.float32),
            pltpu.VMEM((_N_CODES, 1), jnp.float32),
        ],
        compiler_params=pltpu.CompilerParams(
            dimension_semantics=("arbitrary",)),
    )(x2d, w1, b1, g1, be1, w2, b2, codebook)
    return z, tok.reshape(_ROWS)


@functools.cache
def _make_sc_gather():
    mesh = plsc.VectorSubcoreMesh(core_axis_name="c", subcore_axis_name="s")

    nb = 4
    br = _BPW // nb

    @functools.partial(
        pl.kernel,
        mesh=mesh,
        out_type=jax.ShapeDtypeStruct((_ROWS, _CODE_DIM), jnp.float32),
        scratch_types=[
            pltpu.VMEM((_BPW,), jnp.int32),
            pltpu.VMEM((_BPW, _CODE_DIM), jnp.float32),
            pltpu.SemaphoreType.DMA,
            pltpu.SemaphoreType.DMA,
            pltpu.SemaphoreType.DMA,
            pltpu.SemaphoreType.DMA,
        ],
    )
    def _sc_gather(cb_hbm, idx_hbm, out_hbm, idx_v, rows_v, s0, s1, s2, s3):
        sems = [s0, s1, s2, s3]
        wid = lax.axis_index("s") * _NC + lax.axis_index("c")
        base = wid * _BPW
        pltpu.sync_copy(idx_hbm.at[pl.ds(base, _BPW)], idx_v)
        copies = [
            pltpu.async_copy(cb_hbm.at[idx_v.at[pl.ds(b * br, br)]],
                             rows_v.at[pl.ds(b * br, br)], sems[b])
            for b in range(nb)
        ]
        for b in range(nb):
            copies[b].wait()
            pltpu.sync_copy(rows_v.at[pl.ds(b * br, br)],
                            out_hbm.at[pl.ds(base + b * br, br)])

    return _sc_gather


def kernel(x, W1, b1, g1, be1, W2, b2, codebook):
    x2d = x.reshape(_ROWS, _IN_DIM)
    z_flat, tokens = _encode_quantize(
        x2d, W1, b1.reshape(1, _HID), g1.reshape(1, _HID),
        be1.reshape(1, _HID), W2, b2.reshape(1, _CODE_DIM), codebook)
    z_q = _make_sc_gather()(codebook, tokens)
    return (tokens.reshape(_B, _N),
            z_q.reshape(_B, _N, _CODE_DIM),
            z_flat.reshape(_B, _N, _CODE_DIM))
